# Initial kernel scaffold; baseline (speedup 1.0000x reference)
import jax
import jax.numpy as jnp
from jax.experimental import pallas as pl


def kernel(q, K, V, sparse_ind, sparse_nnz, gqa_group_size):
    raise NotImplementedError("write your pallas kernel here")


# speedup vs baseline: 47.8668x; 47.8668x over previous
import functools

import jax
import jax.numpy as jnp
from jax import lax
from jax.experimental import pallas as pl
from jax.experimental.pallas import tpu as pltpu, tpu_sc as plsc

C = 128
LANES = 16

_GDN = lax.GatherDimensionNumbers(
    offset_dims=(), collapsed_slice_dims=(0,), start_index_map=(0,))


def _shuffle(x, perm):
    return lax.gather(x, perm[:, None], _GDN, slice_sizes=(1,),
                      mode=lax.GatherScatterMode.PROMISE_IN_BOUNDS)


def _bf_reduce(x, op, perms):
    for perm in perms:
        x = op(x, _shuffle(x, perm))
    return x


@functools.partial(jax.jit, static_argnums=(5, 6, 7))
def _sc_attn(qf, kf, vf, ind, nnz, NP, L, D):
    info = plsc.get_sparse_core_info()
    NC, NS = info.num_cores, info.num_subcores
    NW = NC * NS
    PAIRS_PER_W = NP // NW
    NDG = D // LANES
    NRG = C // LANES
    mesh = plsc.VectorSubcoreMesh(core_axis_name="c", subcore_axis_name="s")

    @functools.partial(
        pl.kernel,
        out_type=jax.ShapeDtypeStruct((NP, D), jnp.float32),
        mesh=mesh,
        scratch_types=[
            pltpu.VMEM((D,), jnp.float32),
            pltpu.VMEM((NP + LANES,), jnp.int32),
            pltpu.VMEM((L,), jnp.int32),
            pltpu.VMEM((C, D), jnp.float32),
            pltpu.VMEM((C, D), jnp.float32),
            pltpu.VMEM((C,), jnp.float32),
            pltpu.VMEM((D,), jnp.float32),
            pltpu.SemaphoreType.DMA,
            pltpu.SemaphoreType.DMA,
        ],
    )
    def attn(qf_h, kf_h, vf_h, ind_h, nnz_h, out_h,
             q_v, nnz_v, idx_v, kbuf, vbuf, sbuf, obuf, ksem, vsem):
        wid = lax.axis_index("s") * NC + lax.axis_index("c")
        lanes = lax.broadcasted_iota(jnp.int32, (LANES,), 0)
        perms = [lanes ^ k for k in (8, 4, 2, 1)]
        pltpu.sync_copy(nnz_h, nnz_v.at[pl.ds(0, NP)])

        def pair_body(i, _):
            pair = i * NW + wid
            n = nnz_v[pl.ds(pair, LANES)][0]
            n = jnp.where(n > 0, n, L)
            nch = (n + C - 1) // C
            pltpu.sync_copy(ind_h.at[pair], idx_v)
            pltpu.sync_copy(qf_h.at[pair], q_v)
            q8 = [q_v[pl.ds(LANES * j, LANES)] for j in range(NDG)]

            def chunk_body(c, carry):
                m_vec, l_vec = carry[0], carry[1]
                o8 = list(carry[2:])
                base = c * C
                idxs = idx_v.at[pl.ds(base, C)]
                kcp = pltpu.async_copy(kf_h.at[idxs], kbuf, ksem)
                vcp = pltpu.async_copy(vf_h.at[idxs], vbuf, vsem)
                kcp.wait()
                vcp.wait()

                def score_g(g, cm):
                    row0 = g * LANES
                    sv = jnp.zeros((LANES,), jnp.float32)
                    for r in range(LANES):
                        row = row0 + r
                        prt = q8[0] * kbuf[row, pl.ds(0, LANES)]
                        for j in range(1, NDG):
                            prt = prt + q8[j] * kbuf[row, pl.ds(LANES * j, LANES)]
                        s = _bf_reduce(prt, jnp.add, perms)
                        sv = jnp.where(lanes == r, s, sv)
                    valid = (base + row0 + lanes) < n
                    sv = jnp.where(valid, sv, jnp.float32(-1e30))
                    sbuf[pl.ds(row0, LANES)] = sv
                    return jnp.maximum(cm, sv)

                cm_vec = lax.fori_loop(
                    0, NRG, score_g, jnp.full((LANES,), -1e30, jnp.float32))
                m_new = jnp.maximum(m_vec, _bf_reduce(cm_vec, jnp.maximum, perms))
                coef = jnp.exp(m_vec - m_new)
                l_vec = l_vec * coef
                o8 = [o * coef for o in o8]

                def accum_g(g, acc):
                    lv = acc[0]
                    oo = list(acc[1:])
                    row0 = g * LANES
                    sv = sbuf[pl.ds(row0, LANES)]
                    pv = jnp.exp(sv - m_new)
                    lv = lv + pv
                    for r in range(LANES):
                        row = row0 + r
                        pr = pv[r]
                        for j in range(NDG):
                            oo[j] = oo[j] + pr * vbuf[row, pl.ds(LANES * j, LANES)]
                    return (lv, *oo)

                acc = lax.fori_loop(0, NRG, accum_g, (l_vec, *o8))
                return (m_new, *acc)

            zero = jnp.zeros((LANES,), jnp.float32)
            init = (jnp.full((LANES,), -1e30, jnp.float32), zero) + (zero,) * NDG
            fin = lax.fori_loop(0, nch, chunk_body, init)
            l_vec = fin[1]
            o8 = list(fin[2:])
            inv = 1.0 / _bf_reduce(l_vec, jnp.add, perms)
            for j in range(NDG):
                obuf[pl.ds(LANES * j, LANES)] = o8[j] * inv
            pltpu.sync_copy(obuf, out_h.at[pair])
            return 0

        lax.fori_loop(0, PAIRS_PER_W, pair_body, 0)

    return attn(qf, kf, vf, ind, nnz)


def kernel(q, K, V, sparse_ind, sparse_nnz, gqa_group_size):
    B, H, _, D = q.shape
    _, HKV, S, _ = K.shape
    L = sparse_ind.shape[-1]
    NP = B * H
    scale = 1.0 / (D ** 0.5)
    qf = (q * scale).reshape(NP, D)
    kf = K.reshape(B * HKV * S, D)
    vf = V.reshape(B * HKV * S, D)
    kvh = jnp.arange(H, dtype=jnp.int32) // gqa_group_size
    rowbase = (jnp.arange(B, dtype=jnp.int32)[:, None] * HKV + kvh[None, :]) * S
    flat_ind = (sparse_ind.astype(jnp.int32) + rowbase[:, :, None]).reshape(NP, L)
    nnzf = sparse_nnz.astype(jnp.int32).reshape(NP)
    out = _sc_attn(qf, kf, vf, flat_ind, nnzf, NP, L, D)
    return out.reshape(B, H, 1, D)


# speedup vs baseline: 74.9380x; 1.5656x over previous
import functools

import jax
import jax.numpy as jnp
from jax import lax
from jax.experimental import pallas as pl
from jax.experimental.pallas import tpu as pltpu, tpu_sc as plsc

C = 128
LANES = 16

_GDN = lax.GatherDimensionNumbers(
    offset_dims=(), collapsed_slice_dims=(0,), start_index_map=(0,))


def _shuffle(x, perm):
    return lax.gather(x, perm[:, None], _GDN, slice_sizes=(1,),
                      mode=lax.GatherScatterMode.PROMISE_IN_BOUNDS)


def _bf_reduce(x, op, perms):
    for perm in perms:
        x = op(x, _shuffle(x, perm))
    return x


@functools.partial(jax.jit, static_argnums=(5, 6, 7))
def _sc_attn(qf, kf, vf, ind, nnz, NP, L, D):
    info = plsc.get_sparse_core_info()
    NC, NS = info.num_cores, info.num_subcores
    NW = NC * NS
    PAIRS_PER_W = NP // NW
    NDG = D // LANES
    NRG = C // LANES
    mesh = plsc.VectorSubcoreMesh(core_axis_name="c", subcore_axis_name="s")

    @functools.partial(
        pl.kernel,
        out_type=jax.ShapeDtypeStruct((NP, D), jnp.float32),
        mesh=mesh,
        scratch_types=[
            pltpu.VMEM((2, D), jnp.float32),
            pltpu.VMEM((NP + LANES,), jnp.int32),
            pltpu.VMEM((2, L), jnp.int32),
            pltpu.VMEM((2, C, D), jnp.float32),
            pltpu.VMEM((2, C, D), jnp.float32),
            pltpu.VMEM((C,), jnp.float32),
            pltpu.VMEM((D,), jnp.float32),
            pltpu.SemaphoreType.DMA((2,)),
            pltpu.SemaphoreType.DMA((2,)),
            pltpu.SemaphoreType.DMA,
            pltpu.SemaphoreType.DMA,
        ],
    )
    def attn(qf_h, kf_h, vf_h, ind_h, nnz_h, out_h,
             q_v, nnz_v, idx_v, kbuf, vbuf, sbuf, obuf, ksem, vsem, isem, qsem):
        wid = lax.axis_index("s") * NC + lax.axis_index("c")
        lanes = lax.broadcasted_iota(jnp.int32, (LANES,), 0)
        perms = [lanes ^ k for k in (8, 4, 2, 1)]
        pltpu.sync_copy(nnz_h, nnz_v.at[pl.ds(0, NP)])

        def n_of(pair):
            n = nnz_v[pl.ds(pair, LANES)][0]
            return jnp.where(n > 0, n, L)

        def issue_chunk(pslot, c, slot):
            idxs = idx_v.at[pslot, pl.ds(c * C, C)]
            pltpu.async_copy(kf_h.at[idxs], kbuf.at[slot], ksem.at[slot])
            pltpu.async_copy(vf_h.at[idxs], vbuf.at[slot], vsem.at[slot])

        pltpu.sync_copy(ind_h.at[wid], idx_v.at[0])
        pltpu.sync_copy(qf_h.at[wid], q_v.at[0])
        issue_chunk(0, 0, 0)

        def pair_body(i, sbase):
            pair = i * NW + wid
            pslot = jnp.bitwise_and(i, 1)
            n = n_of(pair)
            nch = (n + C - 1) // C
            has_next = i + 1 < PAIRS_PER_W

            @pl.when(has_next)
            def _():
                nxt = (i + 1) * NW + wid
                nps = 1 - pslot
                pltpu.async_copy(ind_h.at[nxt], idx_v.at[nps], isem)
                pltpu.async_copy(qf_h.at[nxt], q_v.at[nps], qsem)

            q8 = [q_v[pslot, pl.ds(LANES * j, LANES)] for j in range(NDG)]

            def chunk_body(c, carry):
                m_vec, l_vec = carry[0], carry[1]
                o8 = list(carry[2:])
                slot = jnp.bitwise_and(sbase + c, 1)

                @pl.when(c + 1 < nch)
                def _():
                    issue_chunk(pslot, c + 1, 1 - slot)

                pltpu.make_async_copy(
                    kf_h.at[idx_v.at[pslot, pl.ds(0, C)]],
                    kbuf.at[slot], ksem.at[slot]).wait()
                pltpu.make_async_copy(
                    vf_h.at[idx_v.at[pslot, pl.ds(0, C)]],
                    vbuf.at[slot], vsem.at[slot]).wait()

                base = c * C

                def score_g(g, cm):
                    row0 = g * LANES
                    sv = jnp.zeros((LANES,), jnp.float32)
                    for r in range(LANES):
                        row = row0 + r
                        prt = q8[0] * kbuf[slot, row, pl.ds(0, LANES)]
                        for j in range(1, NDG):
                            prt = prt + q8[j] * kbuf[slot, row, pl.ds(LANES * j, LANES)]
                        s = _bf_reduce(prt, jnp.add, perms)
                        sv = jnp.where(lanes == r, s, sv)
                    valid = (base + row0 + lanes) < n
                    sv = jnp.where(valid, sv, jnp.float32(-1e30))
                    sbuf[pl.ds(row0, LANES)] = sv
                    return jnp.maximum(cm, sv)

                cm_vec = lax.fori_loop(
                    0, NRG, score_g, jnp.full((LANES,), -1e30, jnp.float32))
                m_new = jnp.maximum(m_vec, _bf_reduce(cm_vec, jnp.maximum, perms))
                coef = jnp.exp(m_vec - m_new)
                l_vec = l_vec * coef
                o8 = [o * coef for o in o8]

                def accum_g(g, acc):
                    lv = acc[0]
                    oo = list(acc[1:])
                    row0 = g * LANES
                    sv = sbuf[pl.ds(row0, LANES)]
                    pv = jnp.exp(sv - m_new)
                    lv = lv + pv
                    for r in range(LANES):
                        row = row0 + r
                        pr = pv[r]
                        for j in range(NDG):
                            oo[j] = oo[j] + pr * vbuf[slot, row, pl.ds(LANES * j, LANES)]
                    return (lv, *oo)

                acc = lax.fori_loop(0, NRG, accum_g, (l_vec, *o8))
                return (m_new, *acc)

            zero = jnp.zeros((LANES,), jnp.float32)
            init = (jnp.full((LANES,), -1e30, jnp.float32), zero) + (zero,) * NDG
            fin = lax.fori_loop(0, nch, chunk_body, init)

            nsbase = jnp.bitwise_and(sbase + nch, 1)

            @pl.when(has_next)
            def _():
                nps = 1 - pslot
                pltpu.make_async_copy(ind_h.at[wid], idx_v.at[nps], isem).wait()
                pltpu.make_async_copy(qf_h.at[wid], q_v.at[nps], qsem).wait()
                issue_chunk(nps, 0, nsbase)

            l_vec = fin[1]
            o8 = list(fin[2:])
            inv = 1.0 / _bf_reduce(l_vec, jnp.add, perms)
            for j in range(NDG):
                obuf[pl.ds(LANES * j, LANES)] = o8[j] * inv
            pltpu.sync_copy(obuf, out_h.at[pair])
            return nsbase

        lax.fori_loop(0, PAIRS_PER_W, pair_body, jnp.int32(0))

    return attn(qf, kf, vf, ind, nnz)


def kernel(q, K, V, sparse_ind, sparse_nnz, gqa_group_size):
    B, H, _, D = q.shape
    _, HKV, S, _ = K.shape
    L = sparse_ind.shape[-1]
    NP = B * H
    scale = 1.0 / (D ** 0.5)
    qf = (q * scale).reshape(NP, D)
    kf = K.reshape(B * HKV * S, D)
    vf = V.reshape(B * HKV * S, D)
    kvh = jnp.arange(H, dtype=jnp.int32) // gqa_group_size
    rowbase = (jnp.arange(B, dtype=jnp.int32)[:, None] * HKV + kvh[None, :]) * S
    flat_ind = (sparse_ind.astype(jnp.int32) + rowbase[:, :, None]).reshape(NP, L)
    nnzf = sparse_nnz.astype(jnp.int32).reshape(NP)
    out = _sc_attn(qf, kf, vf, flat_ind, nnzf, NP, L, D)
    return out.reshape(B, H, 1, D)


# speedup vs baseline: 97.7645x; 1.3046x over previous
import functools

import jax
import jax.numpy as jnp
from jax import lax
from jax.experimental import pallas as pl
from jax.experimental.pallas import tpu as pltpu, tpu_sc as plsc

C = 128
LANES = 16

_GDN = lax.GatherDimensionNumbers(
    offset_dims=(), collapsed_slice_dims=(0,), start_index_map=(0,))


def _shuffle(x, perm):
    return lax.gather(x, perm[:, None], _GDN, slice_sizes=(1,),
                      mode=lax.GatherScatterMode.PROMISE_IN_BOUNDS)


def _bf_reduce(x, op, perms):
    for perm in perms:
        x = op(x, _shuffle(x, perm))
    return x


def _combine(a, b, k, lanes):
    mk = (lanes & k) == 0
    perm = lanes ^ k
    lo = jnp.where(mk, a, _shuffle(b, perm))
    hi = jnp.where(mk, _shuffle(a, perm), b)
    return lo + hi


@functools.partial(jax.jit, static_argnums=(6, 7, 8))
def _sc_attn(qf, kf, vf, ind, nnz, asg, NP, L, D):
    info = plsc.get_sparse_core_info()
    NC, NS = info.num_cores, info.num_subcores
    NW = NC * NS
    PAIRS_PER_W = NP // NW
    NDG = D // LANES
    NRG = C // LANES
    mesh = plsc.VectorSubcoreMesh(core_axis_name="c", subcore_axis_name="s")

    @functools.partial(
        pl.kernel,
        out_type=jax.ShapeDtypeStruct((NP, D), jnp.float32),
        mesh=mesh,
        scratch_types=[
            pltpu.VMEM((2, D), jnp.float32),
            pltpu.VMEM((NP + LANES,), jnp.int32),
            pltpu.VMEM((NP + LANES,), jnp.int32),
            pltpu.VMEM((2, L), jnp.int32),
            pltpu.VMEM((2, C, D), jnp.float32),
            pltpu.VMEM((2, C, D), jnp.float32),
            pltpu.VMEM((C,), jnp.float32),
            pltpu.VMEM((D,), jnp.float32),
            pltpu.SemaphoreType.DMA((2,)),
            pltpu.SemaphoreType.DMA((2,)),
            pltpu.SemaphoreType.DMA,
            pltpu.SemaphoreType.DMA,
        ],
    )
    def attn(qf_h, kf_h, vf_h, ind_h, nnz_h, asg_h, out_h,
             q_v, nnz_v, asg_v, idx_v, kbuf, vbuf, sbuf, obuf,
             ksem, vsem, isem, qsem):
        wid = lax.axis_index("s") * NC + lax.axis_index("c")
        lanes = lax.broadcasted_iota(jnp.int32, (LANES,), 0)
        perms = [lanes ^ k for k in (8, 4, 2, 1)]
        pltpu.sync_copy(nnz_h, nnz_v.at[pl.ds(0, NP)])
        pltpu.sync_copy(asg_h, asg_v.at[pl.ds(0, NP)])

        def n_of(pair):
            n = nnz_v[pl.ds(pair, LANES)][0]
            return jnp.where(n > 0, n, L)

        def pair_of(i):
            return asg_v[pl.ds(wid * PAIRS_PER_W + i, LANES)][0]

        def issue_chunk(pslot, c, slot):
            idxs = idx_v.at[pslot, pl.ds(c * C, C)]
            pltpu.async_copy(kf_h.at[idxs], kbuf.at[slot], ksem.at[slot])
            pltpu.async_copy(vf_h.at[idxs], vbuf.at[slot], vsem.at[slot])

        p0 = pair_of(0)
        pltpu.sync_copy(ind_h.at[p0], idx_v.at[0])
        pltpu.sync_copy(qf_h.at[p0], q_v.at[0])
        issue_chunk(0, 0, 0)

        def pair_body(i, sbase):
            pair = pair_of(i)
            pslot = jnp.bitwise_and(i, 1)
            n = n_of(pair)
            nch = (n + C - 1) // C
            has_next = i + 1 < PAIRS_PER_W

            @pl.when(has_next)
            def _():
                nxt = pair_of(i + 1)
                nps = 1 - pslot
                pltpu.async_copy(ind_h.at[nxt], idx_v.at[nps], isem)
                pltpu.async_copy(qf_h.at[nxt], q_v.at[nps], qsem)

            q8 = [q_v[pslot, pl.ds(LANES * j, LANES)] for j in range(NDG)]

            def chunk_body(c, carry):
                m_vec, l_vec = carry[0], carry[1]
                o8 = list(carry[2:])
                slot = jnp.bitwise_and(sbase + c, 1)

                @pl.when(c + 1 < nch)
                def _():
                    issue_chunk(pslot, c + 1, 1 - slot)

                pltpu.make_async_copy(
                    kf_h.at[idx_v.at[pslot, pl.ds(0, C)]],
                    kbuf.at[slot], ksem.at[slot]).wait()
                pltpu.make_async_copy(
                    vf_h.at[idx_v.at[pslot, pl.ds(0, C)]],
                    vbuf.at[slot], vsem.at[slot]).wait()

                base = c * C

                def score_g(g, cm):
                    row0 = g * LANES
                    stack = []
                    for r in range(LANES):
                        row = row0 + r
                        prods = [q8[j] * kbuf[slot, row, pl.ds(LANES * j, LANES)]
                                 for j in range(NDG)]
                        while len(prods) > 1:
                            prods = [prods[t] + prods[t + 1]
                                     for t in range(0, len(prods), 2)]
                        node, lvl = prods[0], 1
                        while stack and stack[-1][0] == lvl:
                            prev = stack.pop()[1]
                            node = _combine(prev, node, lvl, lanes)
                            lvl *= 2
                        stack.append((lvl, node))
                    sv = stack[0][1]
                    valid = (base + row0 + lanes) < n
                    sv = jnp.where(valid, sv, jnp.float32(-1e30))
                    sbuf[pl.ds(row0, LANES)] = sv
                    return jnp.maximum(cm, sv)

                ng = jnp.minimum(NRG, (n - base + LANES - 1) // LANES)
                cm_vec = lax.fori_loop(
                    0, ng, score_g, jnp.full((LANES,), -1e30, jnp.float32))
                m_new = jnp.maximum(m_vec, _bf_reduce(cm_vec, jnp.maximum, perms))
                coef = jnp.exp(m_vec - m_new)
                l_vec = l_vec * coef
                o8 = [o * coef for o in o8]

                def accum_g(g, acc):
                    lv = acc[0]
                    oo = list(acc[1:])
                    row0 = g * LANES
                    sv = sbuf[pl.ds(row0, LANES)]
                    pv = jnp.exp(sv - m_new)
                    lv = lv + pv
                    loc = [[None] * NDG, [None] * NDG]
                    for r in range(LANES):
                        row = row0 + r
                        pr = _shuffle(pv, jnp.full((LANES,), r, jnp.int32))
                        tgt = loc[r & 1]
                        for j in range(NDG):
                            t = pr * vbuf[slot, row, pl.ds(LANES * j, LANES)]
                            tgt[j] = t if tgt[j] is None else tgt[j] + t
                    oo = [oo[j] + (loc[0][j] + loc[1][j]) for j in range(NDG)]
                    return (lv, *oo)

                acc = lax.fori_loop(0, ng, accum_g, (l_vec, *o8))
                return (m_new, *acc)

            zero = jnp.zeros((LANES,), jnp.float32)
            init = (jnp.full((LANES,), -1e30, jnp.float32), zero) + (zero,) * NDG
            fin = lax.fori_loop(0, nch, chunk_body, init)

            nsbase = jnp.bitwise_and(sbase + nch, 1)

            @pl.when(has_next)
            def _():
                nps = 1 - pslot
                pltpu.make_async_copy(ind_h.at[wid], idx_v.at[nps], isem).wait()
                pltpu.make_async_copy(qf_h.at[wid], q_v.at[nps], qsem).wait()
                issue_chunk(nps, 0, nsbase)

            l_vec = fin[1]
            o8 = list(fin[2:])
            inv = 1.0 / _bf_reduce(l_vec, jnp.add, perms)
            for j in range(NDG):
                obuf[pl.ds(LANES * j, LANES)] = o8[j] * inv
            pltpu.sync_copy(obuf, out_h.at[pair])
            return nsbase

        lax.fori_loop(0, PAIRS_PER_W, pair_body, jnp.int32(0))

    return attn(qf, kf, vf, ind, nnz, asg)


def kernel(q, K, V, sparse_ind, sparse_nnz, gqa_group_size):
    B, H, _, D = q.shape
    _, HKV, S, _ = K.shape
    L = sparse_ind.shape[-1]
    NP = B * H
    scale = 1.0 / (D ** 0.5)
    qf = (q * scale).reshape(NP, D)
    kf = K.reshape(B * HKV * S, D)
    vf = V.reshape(B * HKV * S, D)
    kvh = jnp.arange(H, dtype=jnp.int32) // gqa_group_size
    rowbase = (jnp.arange(B, dtype=jnp.int32)[:, None] * HKV + kvh[None, :]) * S
    flat_ind = (sparse_ind.astype(jnp.int32) + rowbase[:, :, None]).reshape(NP, L)
    nnzf = sparse_nnz.astype(jnp.int32).reshape(NP)
    info = plsc.get_sparse_core_info()
    NW = info.num_cores * info.num_subcores
    R = NP // NW
    n_eff = jnp.where(nnzf > 0, nnzf, L)
    order = jnp.argsort(-n_eff).astype(jnp.int32).reshape(R, NW)
    snake = jnp.where((jnp.arange(R) % 2 == 0)[:, None], order, order[:, ::-1])
    asg = snake.T.reshape(NP)
    out = _sc_attn(qf, kf, vf, flat_ind, nnzf, asg, NP, L, D)
    return out.reshape(B, H, 1, D)
